# Initial kernel scaffold; baseline (speedup 1.0000x reference)
import functools

import jax
import jax.numpy as jnp
from jax import lax
from jax.experimental import pallas as pl
from jax.experimental.pallas import tpu as pltpu
from jax.experimental.pallas import tpu_sc as plsc

NC = 2
NS = 16
NW = NC * NS
L = 16
CH = 128


def _k1_scalars(x_ref, er_ref, wa_ref, wa2_ref, ssrc_ref, sdst_ref, srel_ref):
    d = x_ref.shape[1]
    w = jnp.dot(wa2_ref[:], wa_ref[:], preferred_element_type=jnp.float32)
    ssrc_ref[:] = jnp.sum(x_ref[:] * w[:, :d], axis=1, keepdims=True)
    sdst_ref[:] = jnp.sum(x_ref[:] * w[:, d:2 * d], axis=1, keepdims=True)
    srel_ref[:] = jnp.sum(er_ref[:] * w[:, 2 * d:], axis=1, keepdims=True)


def _k4_combine(p_ref, o_ref):
    o_ref[:] = p_ref[0] + p_ref[1]


def kernel(x, edge_index, edge_type, pm_pd, emb_rel, W_att, W_att2):
    del pm_pd
    n, d = x.shape
    e_total = edge_type.shape[0]
    r = emb_rel.shape[0]

    rows = e_total // CH
    rpw = pl.cdiv(rows, NW)
    rows_pad = rpw * NW
    npad = pl.cdiv(n, NS * L) * NS * L
    nband = npad // NS
    hband = n // NS
    rpad = pl.cdiv(r, L) * L * 2

    er_p = jnp.zeros((rpad, d), jnp.float32).at[:r].set(emb_rel)
    s_src, s_dst, s_rel = pl.pallas_call(
        _k1_scalars,
        out_shape=(
            jax.ShapeDtypeStruct((n, 1), jnp.float32),
            jax.ShapeDtypeStruct((n, 1), jnp.float32),
            jax.ShapeDtypeStruct((rpad, 1), jnp.float32),
        ),
    )(x, er_p, W_att, W_att2)
    s_src = s_src.reshape(n)
    s_dst = s_dst.reshape(n)
    s_rel = s_rel.reshape(rpad)

    pad_e = rows_pad * CH - e_total
    src2d = jnp.pad(edge_index[0], (0, pad_e)).reshape(rows_pad, CH)
    dst2d = jnp.pad(edge_index[1], (0, pad_e)).reshape(rows_pad, CH)
    et2d = jnp.pad(edge_type, (0, pad_e)).reshape(rows_pad, CH)

    mesh = plsc.VectorSubcoreMesh(core_axis_name="c", subcore_axis_name="s")

    @functools.partial(
        pl.kernel,
        out_type=(
            jax.ShapeDtypeStruct((rows_pad, CH), jnp.float32),
            jax.ShapeDtypeStruct((NC, npad), jnp.float32),
        ),
        mesh=mesh,
        scratch_types=[
            pltpu.VMEM((n,), jnp.float32),
            pltpu.VMEM((n,), jnp.float32),
            pltpu.VMEM((rpad,), jnp.float32),
            pltpu.VMEM((rpw, CH), jnp.int32),
            pltpu.VMEM((rpw, CH), jnp.int32),
            pltpu.VMEM((rpw, CH), jnp.int32),
            pltpu.VMEM((rpw, CH), jnp.float32),
            pltpu.VMEM((npad,), jnp.float32),
            pltpu.VMEM_SHARED((NS, npad), jnp.float32),
            pltpu.VMEM((NS, nband), jnp.float32),
            pltpu.VMEM((nband,), jnp.float32),
        ],
    )
    def _k2(ssrc_hbm, sdst_hbm, srel_hbm, src_hbm, dst_hbm, et_hbm,
            ee_hbm, den_hbm,
            ssrc_v, sdst_v, srel_v, srcb, dstb, etb, eeb, denp,
            stage_sp, redv, outv):
        c = lax.axis_index("c")
        s = lax.axis_index("s")
        base = (c * NS + s) * rpw

        pltpu.sync_copy(ssrc_hbm, ssrc_v)
        pltpu.sync_copy(sdst_hbm, sdst_v)
        pltpu.sync_copy(srel_hbm, srel_v)
        pltpu.sync_copy(src_hbm.at[pl.ds(base, rpw), :], srcb)
        pltpu.sync_copy(dst_hbm.at[pl.ds(base, rpw), :], dstb)
        pltpu.sync_copy(et_hbm.at[pl.ds(base, rpw), :], etb)

        def zero_body(i, carry):
            denp[pl.ds(i * L, L)] = jnp.zeros((L,), jnp.float32)
            return carry
        lax.fori_loop(0, npad // L, zero_body, None)

        def chunk_body(j, carry):
            @pl.when(base + j < rows)
            def _():
                def grp_body(i, c2):
                    sl = pl.ds(i * L, L)
                    si = srcb[j, sl]
                    di = dstb[j, sl]
                    ti = etb[j, sl]
                    ev = (plsc.load_gather(ssrc_v, [si])
                          + plsc.load_gather(sdst_v, [di])
                          + plsc.load_gather(srel_v, [ti]))
                    ev = jnp.where(ev >= 0.0, ev, ev * jnp.float32(0.01))
                    ee = jnp.exp(ev)
                    eeb[j, sl] = ee
                    plsc.addupdate_scatter(denp, [di], ee)
                    return c2
                lax.fori_loop(0, CH // L, grp_body, None)
            return carry
        lax.fori_loop(0, rpw, chunk_body, None)

        pltpu.sync_copy(eeb, ee_hbm.at[pl.ds(base, rpw), :])

        pltpu.sync_copy(denp, stage_sp.at[s])
        plsc.subcore_barrier()
        pltpu.sync_copy(stage_sp.at[:, pl.ds(s * nband, nband)], redv)

        def red_body(i, carry):
            sl = pl.ds(i * L, L)
            acc = redv[0, sl]
            for k in range(1, NS):
                acc = acc + redv[k, sl]
            outv[sl] = acc
            return carry
        lax.fori_loop(0, nband // L, red_body, None)
        pltpu.sync_copy(outv, den_hbm.at[c, pl.ds(s * nband, nband)])

    ee2d, den_part = _k2(s_src, s_dst, s_rel, src2d, dst2d, et2d)

    @functools.partial(
        pl.kernel,
        out_type=jax.ShapeDtypeStruct((NC, n, d), jnp.float32),
        mesh=mesh,
        scratch_types=[
            pltpu.VMEM((npad,), jnp.float32),
            pltpu.VMEM((npad,), jnp.float32),
            pltpu.VMEM((rpw, CH), jnp.int32),
            pltpu.VMEM((rpw, CH), jnp.int32),
            pltpu.VMEM((rpw, CH), jnp.float32),
            pltpu.VMEM((CH,), jnp.float32),
            pltpu.VMEM((CH, d), jnp.float32),
            pltpu.VMEM_SHARED((n, d), jnp.float32),
            pltpu.SemaphoreType.DMA,
        ],
    )
    def _k3(x_hbm, src_hbm, dst_hbm, ee_hbm, den_hbm, out_hbm,
            denb, dent, srcb, dstb, eeb, alb, rowb, hout_sp, sem):
        c = lax.axis_index("c")
        s = lax.axis_index("s")
        base = (c * NS + s) * rpw

        pltpu.sync_copy(den_hbm.at[0], denb)
        pltpu.sync_copy(den_hbm.at[1], dent)

        def add_body(i, carry):
            sl = pl.ds(i * L, L)
            denb[sl] = denb[sl] + dent[sl]
            return carry
        lax.fori_loop(0, npad // L, add_body, None)

        def zrow_body(i, carry):
            rowb[i // (d // L), pl.ds((i % (d // L)) * L, L)] = (
                jnp.zeros((L,), jnp.float32))
            return carry
        lax.fori_loop(0, CH * d // (L * L), zrow_body, None)
        band = s * hband
        nfull = hband // CH
        for k in range(nfull):
            pltpu.sync_copy(rowb, hout_sp.at[pl.ds(band + k * CH, CH), :])
        rem = hband - nfull * CH
        if rem:
            pltpu.sync_copy(rowb.at[pl.ds(0, rem), :],
                            hout_sp.at[pl.ds(band + nfull * CH, rem), :])

        pltpu.sync_copy(src_hbm.at[pl.ds(base, rpw), :], srcb)
        pltpu.sync_copy(dst_hbm.at[pl.ds(base, rpw), :], dstb)
        pltpu.sync_copy(ee_hbm.at[pl.ds(base, rpw), :], eeb)
        plsc.subcore_barrier()

        def chunk_body(j, carry):
            @pl.when(base + j < rows)
            def _():
                def alpha_body(i, c2):
                    sl = pl.ds(i * L, L)
                    di = dstb[j, sl]
                    den = plsc.load_gather(denb, [di])
                    alb[sl] = eeb[j, sl] / den
                    return c2
                lax.fori_loop(0, CH // L, alpha_body, None)

                pltpu.async_copy(x_hbm.at[srcb.at[j]], rowb, sem).wait()

                def scale_body(rr, c2):
                    spl = plsc.load_gather(
                        alb, [jnp.full((L,), rr, jnp.int32)])
                    for g in range(d // L):
                        sl = pl.ds(g * L, L)
                        rowb[rr, sl] = rowb[rr, sl] * spl
                    return c2
                lax.fori_loop(0, CH, scale_body, None)

                pltpu.sync_copy(rowb, hout_sp.at[dstb.at[j]], add=True)
            return carry
        lax.fori_loop(0, rpw, chunk_body, None)

        plsc.subcore_barrier()
        pltpu.sync_copy(hout_sp.at[pl.ds(band, hband), :],
                        out_hbm.at[c, pl.ds(band, hband), :])

    h_part = _k3(x, src2d, dst2d, ee2d, den_part)

    h_out = pl.pallas_call(
        _k4_combine,
        out_shape=jax.ShapeDtypeStruct((n, d), jnp.float32),
    )(h_part)
    return h_out


# speedup vs baseline: 21.8359x; 21.8359x over previous
import functools

import jax
import jax.numpy as jnp
from jax import lax
from jax.experimental import pallas as pl
from jax.experimental.pallas import tpu as pltpu
from jax.experimental.pallas import tpu_sc as plsc

NC = 2
NS = 16
NW = NC * NS
L = 16
CH = 128


def _k1_scalars(x_ref, er_ref, wa_ref, wa2_ref, ssrc_ref, sdst_ref, srel_ref):
    d = x_ref.shape[1]
    w = jnp.dot(wa2_ref[:], wa_ref[:], preferred_element_type=jnp.float32)
    ssrc_ref[:] = jnp.sum(x_ref[:] * w[:, :d], axis=1, keepdims=True)
    sdst_ref[:] = jnp.sum(x_ref[:] * w[:, d:2 * d], axis=1, keepdims=True)
    srel_ref[:] = jnp.sum(er_ref[:] * w[:, 2 * d:], axis=1, keepdims=True)


def _k4_combine(p_ref, o_ref):
    nn = o_ref.shape[0]
    o_ref[:] = p_ref[0, :nn, :] + p_ref[1, :nn, :]


def kernel(x, edge_index, edge_type, pm_pd, emb_rel, W_att, W_att2):
    del pm_pd
    n, d = x.shape
    e_total = edge_type.shape[0]
    r = emb_rel.shape[0]

    rows = e_total // CH
    rpw = pl.cdiv(pl.cdiv(rows, NW), 8) * 8
    rows_pad = rpw * NW
    npad = pl.cdiv(n, NS * L) * NS * L
    nband = npad // NS
    hband = npad // NS
    rpad = pl.cdiv(r, L) * L * 2

    er_p = jnp.zeros((rpad, d), jnp.float32).at[:r].set(emb_rel)
    s_src, s_dst, s_rel = pl.pallas_call(
        _k1_scalars,
        out_shape=(
            jax.ShapeDtypeStruct((n, 1), jnp.float32),
            jax.ShapeDtypeStruct((n, 1), jnp.float32),
            jax.ShapeDtypeStruct((rpad, 1), jnp.float32),
        ),
    )(x, er_p, W_att, W_att2)
    s_src = s_src.reshape(n)
    s_dst = s_dst.reshape(n)
    s_rel = s_rel.reshape(rpad)

    pad_e = rows_pad * CH - e_total
    src2d = jnp.pad(edge_index[0], (0, pad_e)).reshape(rows_pad, CH)
    dst2d = jnp.pad(edge_index[1], (0, pad_e)).reshape(rows_pad, CH)
    et2d = jnp.pad(edge_type, (0, pad_e)).reshape(rows_pad, CH)

    mesh = plsc.VectorSubcoreMesh(core_axis_name="c", subcore_axis_name="s")

    @functools.partial(
        pl.kernel,
        out_type=(
            jax.ShapeDtypeStruct((rows_pad, CH), jnp.float32),
            jax.ShapeDtypeStruct((NC * npad,), jnp.float32),
        ),
        mesh=mesh,
        compiler_params=pltpu.CompilerParams(needs_layout_passes=False),
        scratch_types=[
            pltpu.VMEM((n,), jnp.float32),
            pltpu.VMEM((n,), jnp.float32),
            pltpu.VMEM((rpad,), jnp.float32),
            pltpu.VMEM((rpw, CH), jnp.int32),
            pltpu.VMEM((rpw, CH), jnp.int32),
            pltpu.VMEM((rpw, CH), jnp.int32),
            pltpu.VMEM((rpw, CH), jnp.float32),
            pltpu.VMEM((npad,), jnp.float32),
            pltpu.VMEM_SHARED((NS, npad), jnp.float32),
            pltpu.VMEM((NS, nband), jnp.float32),
            pltpu.VMEM((nband,), jnp.float32),
        ],
    )
    def _k2(ssrc_hbm, sdst_hbm, srel_hbm, src_hbm, dst_hbm, et_hbm,
            ee_hbm, den_hbm,
            ssrc_v, sdst_v, srel_v, srcb, dstb, etb, eeb, denp,
            stage_sp, redv, outv):
        c = lax.axis_index("c")
        s = lax.axis_index("s")
        base = (c * NS + s) * rpw

        pltpu.sync_copy(ssrc_hbm, ssrc_v)
        pltpu.sync_copy(sdst_hbm, sdst_v)
        pltpu.sync_copy(srel_hbm, srel_v)
        pltpu.sync_copy(src_hbm.at[pl.ds(base, rpw), :], srcb)
        pltpu.sync_copy(dst_hbm.at[pl.ds(base, rpw), :], dstb)
        pltpu.sync_copy(et_hbm.at[pl.ds(base, rpw), :], etb)

        def zero_body(i, carry):
            denp[pl.ds(i * L, L)] = jnp.zeros((L,), jnp.float32)
            return carry
        lax.fori_loop(0, npad // L, zero_body, None)

        def chunk_body(j, carry):
            @pl.when(base + j < rows)
            def _():
                def grp_body(i, c2):
                    sl = pl.ds(i * L, L)
                    si = srcb[j, sl]
                    di = dstb[j, sl]
                    ti = etb[j, sl]
                    ev = (plsc.load_gather(ssrc_v, [si])
                          + plsc.load_gather(sdst_v, [di])
                          + plsc.load_gather(srel_v, [ti]))
                    ev = jnp.where(ev >= 0.0, ev, ev * jnp.float32(0.01))
                    ee = jnp.exp(ev)
                    eeb[j, sl] = ee
                    plsc.addupdate_scatter(denp, [di], ee)
                    return c2
                lax.fori_loop(0, CH // L, grp_body, None)
            return carry
        lax.fori_loop(0, rpw, chunk_body, None)

        pltpu.sync_copy(eeb, ee_hbm.at[pl.ds(base, rpw), :])

        pltpu.sync_copy(denp, stage_sp.at[s])
        plsc.subcore_barrier()
        pltpu.sync_copy(stage_sp.at[:, pl.ds(s * nband, nband)], redv)

        def red_body(i, carry):
            sl = pl.ds(i * L, L)
            acc = redv[0, sl]
            for k in range(1, NS):
                acc = acc + redv[k, sl]
            outv[sl] = acc
            return carry
        lax.fori_loop(0, nband // L, red_body, None)
        pltpu.sync_copy(outv, den_hbm.at[pl.ds(c * npad + s * nband, nband)])

    ee2d, den_part = _k2(s_src, s_dst, s_rel, src2d, dst2d, et2d)

    BB = 16
    nblk = rpw // BB

    @functools.partial(
        pl.kernel,
        out_type=jax.ShapeDtypeStruct((NC, n, d), jnp.float32),
        mesh=mesh,
        compiler_params=pltpu.CompilerParams(needs_layout_passes=False),
        scratch_types=[
            pltpu.VMEM((npad,), jnp.float32),
            pltpu.VMEM((npad,), jnp.float32),
            pltpu.VMEM((BB, CH), jnp.int32),
            pltpu.VMEM((BB, CH), jnp.int32),
            pltpu.VMEM((BB, CH), jnp.float32),
            pltpu.VMEM((CH,), jnp.float32),
            pltpu.VMEM((CH, d), jnp.float32),
            pltpu.VMEM_SHARED((n, d), jnp.float32),
            pltpu.SemaphoreType.DMA,
        ],
    )
    def _k3(x_hbm, src_hbm, dst_hbm, ee_hbm, den_hbm, out_hbm,
            denb, dent, srcb, dstb, eeb, alb, rowb, hout_sp, sem):
        c = lax.axis_index("c")
        s = lax.axis_index("s")
        base = (c * NS + s) * rpw

        pltpu.sync_copy(den_hbm.at[pl.ds(0, npad)], denb)
        pltpu.sync_copy(den_hbm.at[pl.ds(npad, npad)], dent)

        def add_body(i, carry):
            sl = pl.ds(i * L, L)
            denb[sl] = denb[sl] + dent[sl]
            return carry
        lax.fori_loop(0, npad // L, add_body, None)

        def zrow_body(i, carry):
            rowb[i // (d // L), pl.ds((i % (d // L)) * L, L)] = (
                jnp.zeros((L,), jnp.float32))
            return carry
        lax.fori_loop(0, CH * d // (L * L), zrow_body, None)
        band = pl.multiple_of(jnp.minimum(s * hband, n - hband), 16)
        nfull = hband // CH
        for k in range(nfull):
            pltpu.sync_copy(rowb, hout_sp.at[pl.ds(band + k * CH, CH), :])
        rem = hband - nfull * CH
        if rem:
            pltpu.sync_copy(rowb.at[pl.ds(0, rem), :],
                            hout_sp.at[pl.ds(band + nfull * CH, rem), :])
        plsc.subcore_barrier()

        def blk_body(b, carry):
            brow = base + b * BB
            pltpu.sync_copy(src_hbm.at[pl.ds(brow, BB), :], srcb)
            pltpu.sync_copy(dst_hbm.at[pl.ds(brow, BB), :], dstb)
            pltpu.sync_copy(ee_hbm.at[pl.ds(brow, BB), :], eeb)

            def chunk_body(j, carry2):
                @pl.when(brow + j < rows)
                def _():
                    def alpha_body(i, c2):
                        sl = pl.ds(i * L, L)
                        di = dstb[j, sl]
                        den = plsc.load_gather(denb, [di])
                        alb[sl] = eeb[j, sl] / den
                        return c2
                    lax.fori_loop(0, CH // L, alpha_body, None)

                    pltpu.async_copy(x_hbm.at[srcb.at[j]], rowb, sem).wait()

                    def scale_body(rr, c2):
                        spl = plsc.load_gather(
                            alb, [jnp.full((L,), rr, jnp.int32)])
                        for g in range(d // L):
                            sl = pl.ds(g * L, L)
                            rowb[rr, sl] = rowb[rr, sl] * spl
                        return c2
                    lax.fori_loop(0, CH, scale_body, None)

                    pltpu.sync_copy(rowb, hout_sp.at[dstb.at[j]], add=True)
                return carry2
            lax.fori_loop(0, BB, chunk_body, None)
            return carry
        lax.fori_loop(0, nblk, blk_body, None)

        plsc.subcore_barrier()
        pltpu.sync_copy(hout_sp.at[pl.ds(band, hband), :],
                        out_hbm.at[c, pl.ds(band, hband), :])

    h_part = _k3(x, src2d, dst2d, ee2d, den_part)

    h_out = pl.pallas_call(
        _k4_combine,
        out_shape=jax.ShapeDtypeStruct((n, d), jnp.float32),
    )(h_part)
    return h_out


# speedup vs baseline: 22.9196x; 1.0496x over previous
import functools

import jax
import jax.numpy as jnp
from jax import lax
from jax.experimental import pallas as pl
from jax.experimental.pallas import tpu as pltpu
from jax.experimental.pallas import tpu_sc as plsc

NC = 2
NS = 16
NW = NC * NS
L = 16
CH = 128


def _k1_scalars(x_ref, er_ref, wa_ref, wa2_ref, ssrc_ref, sdst_ref, srel_ref):
    d = x_ref.shape[1]
    w = jnp.dot(wa2_ref[:], wa_ref[:], preferred_element_type=jnp.float32)
    ssrc_ref[:] = jnp.sum(x_ref[:] * w[:, :d], axis=1, keepdims=True)
    sdst_ref[:] = jnp.sum(x_ref[:] * w[:, d:2 * d], axis=1, keepdims=True)
    srel_ref[:] = jnp.sum(er_ref[:] * w[:, 2 * d:], axis=1, keepdims=True)


def _k4_combine(p_ref, o_ref):
    nn = o_ref.shape[0]
    o_ref[:] = p_ref[0, :nn, :] + p_ref[1, :nn, :]


def kernel(x, edge_index, edge_type, pm_pd, emb_rel, W_att, W_att2):
    del pm_pd
    n, d = x.shape
    e_total = edge_type.shape[0]
    r = emb_rel.shape[0]

    rows = e_total // CH
    rpw = pl.cdiv(pl.cdiv(rows, NW), 8) * 8
    rows_pad = rpw * NW
    npad = pl.cdiv(n, NS * L) * NS * L
    nband = npad // NS
    hband = npad // NS
    rpad = pl.cdiv(r, L) * L * 2

    er_p = jnp.zeros((rpad, d), jnp.float32).at[:r].set(emb_rel)
    s_src, s_dst, s_rel = pl.pallas_call(
        _k1_scalars,
        out_shape=(
            jax.ShapeDtypeStruct((n, 1), jnp.float32),
            jax.ShapeDtypeStruct((n, 1), jnp.float32),
            jax.ShapeDtypeStruct((rpad, 1), jnp.float32),
        ),
    )(x, er_p, W_att, W_att2)
    s_src = s_src.reshape(n)
    s_dst = s_dst.reshape(n)
    s_rel = s_rel.reshape(rpad)

    pad_e = rows_pad * CH - e_total
    src2d = jnp.pad(edge_index[0], (0, pad_e)).reshape(rows_pad, CH)
    dst2d = jnp.pad(edge_index[1], (0, pad_e)).reshape(rows_pad, CH)
    et2d = jnp.pad(edge_type, (0, pad_e)).reshape(rows_pad, CH)

    mesh = plsc.VectorSubcoreMesh(core_axis_name="c", subcore_axis_name="s")

    @functools.partial(
        pl.kernel,
        out_type=(
            jax.ShapeDtypeStruct((rows_pad, CH), jnp.float32),
            jax.ShapeDtypeStruct((NC * npad,), jnp.float32),
        ),
        mesh=mesh,
        compiler_params=pltpu.CompilerParams(needs_layout_passes=False),
        scratch_types=[
            pltpu.VMEM((n,), jnp.float32),
            pltpu.VMEM((n,), jnp.float32),
            pltpu.VMEM((rpad,), jnp.float32),
            pltpu.VMEM((rpw, CH), jnp.int32),
            pltpu.VMEM((rpw, CH), jnp.int32),
            pltpu.VMEM((rpw, CH), jnp.int32),
            pltpu.VMEM((rpw, CH), jnp.float32),
            pltpu.VMEM((npad,), jnp.float32),
            pltpu.VMEM_SHARED((NS, npad), jnp.float32),
            pltpu.VMEM((NS, nband), jnp.float32),
            pltpu.VMEM((nband,), jnp.float32),
        ],
    )
    def _k2(ssrc_hbm, sdst_hbm, srel_hbm, src_hbm, dst_hbm, et_hbm,
            ee_hbm, den_hbm,
            ssrc_v, sdst_v, srel_v, srcb, dstb, etb, eeb, denp,
            stage_sp, redv, outv):
        c = lax.axis_index("c")
        s = lax.axis_index("s")
        base = (c * NS + s) * rpw

        pltpu.sync_copy(ssrc_hbm, ssrc_v)
        pltpu.sync_copy(sdst_hbm, sdst_v)
        pltpu.sync_copy(srel_hbm, srel_v)
        pltpu.sync_copy(src_hbm.at[pl.ds(base, rpw), :], srcb)
        pltpu.sync_copy(dst_hbm.at[pl.ds(base, rpw), :], dstb)
        pltpu.sync_copy(et_hbm.at[pl.ds(base, rpw), :], etb)

        def zero_body(i, carry):
            denp[pl.ds(i * L, L)] = jnp.zeros((L,), jnp.float32)
            return carry
        lax.fori_loop(0, npad // L, zero_body, None)

        def chunk_body(j, carry):
            @pl.when(base + j < rows)
            def _():
                def grp_body(i, c2):
                    sl = pl.ds(i * L, L)
                    si = srcb[j, sl]
                    di = dstb[j, sl]
                    ti = etb[j, sl]
                    ev = (plsc.load_gather(ssrc_v, [si])
                          + plsc.load_gather(sdst_v, [di])
                          + plsc.load_gather(srel_v, [ti]))
                    ev = jnp.where(ev >= 0.0, ev, ev * jnp.float32(0.01))
                    ee = jnp.exp(ev)
                    eeb[j, sl] = ee
                    plsc.addupdate_scatter(denp, [di], ee)
                    return c2
                lax.fori_loop(0, CH // L, grp_body, None)
            return carry
        lax.fori_loop(0, rpw, chunk_body, None)

        pltpu.sync_copy(eeb, ee_hbm.at[pl.ds(base, rpw), :])

        pltpu.sync_copy(denp, stage_sp.at[s])
        plsc.subcore_barrier()
        pltpu.sync_copy(stage_sp.at[:, pl.ds(s * nband, nband)], redv)

        def red_body(i, carry):
            sl = pl.ds(i * L, L)
            acc = redv[0, sl]
            for k in range(1, NS):
                acc = acc + redv[k, sl]
            outv[sl] = acc
            return carry
        lax.fori_loop(0, nband // L, red_body, None)
        pltpu.sync_copy(outv, den_hbm.at[pl.ds(c * npad + s * nband, nband)])

    ee2d, den_part = _k2(s_src, s_dst, s_rel, src2d, dst2d, et2d)

    BB = 16
    nblk = rpw // BB

    @functools.partial(
        pl.kernel,
        out_type=jax.ShapeDtypeStruct((NC, n, d), jnp.float32),
        mesh=mesh,
        compiler_params=pltpu.CompilerParams(needs_layout_passes=False),
        scratch_types=[
            pltpu.VMEM((npad,), jnp.float32),
            pltpu.VMEM((npad,), jnp.float32),
            pltpu.VMEM((BB, CH), jnp.int32),
            pltpu.VMEM((BB, CH), jnp.int32),
            pltpu.VMEM((BB, CH), jnp.float32),
            pltpu.VMEM((CH,), jnp.float32),
            pltpu.VMEM((CH, d), jnp.float32),
            pltpu.VMEM_SHARED((n, d), jnp.float32),
            pltpu.SemaphoreType.DMA,
        ],
    )
    def _k3(x_hbm, src_hbm, dst_hbm, ee_hbm, den_hbm, out_hbm,
            denb, dent, srcb, dstb, eeb, alb, rowb, hout_sp, sem):
        c = lax.axis_index("c")
        s = lax.axis_index("s")
        base = (c * NS + s) * rpw

        pltpu.sync_copy(den_hbm.at[pl.ds(0, npad)], denb)
        pltpu.sync_copy(den_hbm.at[pl.ds(npad, npad)], dent)

        def add_body(i, carry):
            sl = pl.ds(i * L, L)
            denb[sl] = denb[sl] + dent[sl]
            return carry
        lax.fori_loop(0, npad // L, add_body, None)

        def zrow_body(i, carry):
            rowb[i // (d // L), pl.ds((i % (d // L)) * L, L)] = (
                jnp.zeros((L,), jnp.float32))
            return carry
        lax.fori_loop(0, CH * d // (L * L), zrow_body, None)
        band = pl.multiple_of(jnp.minimum(s * hband, n - hband), 16)
        nfull = hband // CH
        for k in range(nfull):
            pltpu.sync_copy(rowb, hout_sp.at[pl.ds(band + k * CH, CH), :])
        rem = hband - nfull * CH
        if rem:
            pltpu.sync_copy(rowb.at[pl.ds(0, rem), :],
                            hout_sp.at[pl.ds(band + nfull * CH, rem), :])
        plsc.subcore_barrier()

        def blk_body(b, carry):
            brow = base + b * BB
            pltpu.sync_copy(src_hbm.at[pl.ds(brow, BB), :], srcb)
            pltpu.sync_copy(dst_hbm.at[pl.ds(brow, BB), :], dstb)
            pltpu.sync_copy(ee_hbm.at[pl.ds(brow, BB), :], eeb)

            def chunk_body(j, carry2):
                @pl.when(brow + j < rows)
                def _():
                    desc = pltpu.async_copy(x_hbm.at[srcb.at[j]], rowb, sem)

                    def alpha_body(i, c2):
                        sl = pl.ds(i * L, L)
                        di = dstb[j, sl]
                        den = plsc.load_gather(denb, [di])
                        alb[sl] = eeb[j, sl] / den
                        return c2
                    lax.fori_loop(0, CH // L, alpha_body, None)

                    desc.wait()

                    def scale_body(rr, c2):
                        for u in range(2):
                            r2 = rr * 2 + u
                            spl = plsc.load_gather(
                                alb, [jnp.full((L,), r2, jnp.int32)])
                            for g in range(d // L):
                                sl = pl.ds(g * L, L)
                                rowb[r2, sl] = rowb[r2, sl] * spl
                        return c2
                    lax.fori_loop(0, CH // 2, scale_body, None)

                    pltpu.sync_copy(rowb, hout_sp.at[dstb.at[j]], add=True)
                return carry2
            lax.fori_loop(0, BB, chunk_body, None)
            return carry
        lax.fori_loop(0, nblk, blk_body, None)

        plsc.subcore_barrier()
        pltpu.sync_copy(hout_sp.at[pl.ds(band, hband), :],
                        out_hbm.at[c, pl.ds(band, hband), :])

    h_part = _k3(x, src2d, dst2d, ee2d, den_part)

    h_out = pl.pallas_call(
        _k4_combine,
        out_shape=jax.ShapeDtypeStruct((n, d), jnp.float32),
    )(h_part)
    return h_out


# speedup vs baseline: 23.0326x; 1.0049x over previous
import functools

import jax
import jax.numpy as jnp
from jax import lax
from jax.experimental import pallas as pl
from jax.experimental.pallas import tpu as pltpu
from jax.experimental.pallas import tpu_sc as plsc

NC = 2
NS = 16
NW = NC * NS
L = 16
CH = 128


def _k1_scalars(x_ref, er_ref, wa_ref, wa2_ref, ssrc_ref, sdst_ref, srel_ref):
    d = x_ref.shape[1]
    w = jnp.dot(wa2_ref[:], wa_ref[:], preferred_element_type=jnp.float32)
    ssrc_ref[:] = jnp.sum(x_ref[:] * w[:, :d], axis=1, keepdims=True)
    sdst_ref[:] = jnp.sum(x_ref[:] * w[:, d:2 * d], axis=1, keepdims=True)
    srel_ref[:] = jnp.sum(er_ref[:] * w[:, 2 * d:], axis=1, keepdims=True)


def _k4_combine(p_ref, o_ref):
    nn = o_ref.shape[0]
    o_ref[:] = p_ref[0, :nn, :] + p_ref[1, :nn, :]


def kernel(x, edge_index, edge_type, pm_pd, emb_rel, W_att, W_att2):
    del pm_pd
    n, d = x.shape
    e_total = edge_type.shape[0]
    r = emb_rel.shape[0]

    rows = e_total // CH
    rpw = pl.cdiv(pl.cdiv(rows, NW), 8) * 8
    rows_pad = rpw * NW
    npad = pl.cdiv(n, NS * L) * NS * L
    nband = npad // NS
    hband = npad // NS
    rpad = pl.cdiv(r, L) * L * 2

    er_p = jnp.zeros((rpad, d), jnp.float32).at[:r].set(emb_rel)
    s_src, s_dst, s_rel = pl.pallas_call(
        _k1_scalars,
        out_shape=(
            jax.ShapeDtypeStruct((n, 1), jnp.float32),
            jax.ShapeDtypeStruct((n, 1), jnp.float32),
            jax.ShapeDtypeStruct((rpad, 1), jnp.float32),
        ),
    )(x, er_p, W_att, W_att2)
    s_src = s_src.reshape(n)
    s_dst = s_dst.reshape(n)
    s_rel = s_rel.reshape(rpad)

    pad_e = rows_pad * CH - e_total
    src2d = jnp.pad(edge_index[0], (0, pad_e)).reshape(rows_pad, CH)
    dst2d = jnp.pad(edge_index[1], (0, pad_e)).reshape(rows_pad, CH)
    et2d = jnp.pad(edge_type, (0, pad_e)).reshape(rows_pad, CH)

    mesh = plsc.VectorSubcoreMesh(core_axis_name="c", subcore_axis_name="s")

    @functools.partial(
        pl.kernel,
        out_type=(
            jax.ShapeDtypeStruct((rows_pad, CH), jnp.float32),
            jax.ShapeDtypeStruct((NC * npad,), jnp.float32),
        ),
        mesh=mesh,
        compiler_params=pltpu.CompilerParams(needs_layout_passes=False),
        scratch_types=[
            pltpu.VMEM((n,), jnp.float32),
            pltpu.VMEM((n,), jnp.float32),
            pltpu.VMEM((rpad,), jnp.float32),
            pltpu.VMEM((rpw, CH), jnp.int32),
            pltpu.VMEM((rpw, CH), jnp.int32),
            pltpu.VMEM((rpw, CH), jnp.int32),
            pltpu.VMEM((rpw, CH), jnp.float32),
            pltpu.VMEM((npad,), jnp.float32),
            pltpu.VMEM_SHARED((NS, npad), jnp.float32),
            pltpu.VMEM((NS, nband), jnp.float32),
            pltpu.VMEM((nband,), jnp.float32),
        ],
    )
    def _k2(ssrc_hbm, sdst_hbm, srel_hbm, src_hbm, dst_hbm, et_hbm,
            ee_hbm, den_hbm,
            ssrc_v, sdst_v, srel_v, srcb, dstb, etb, eeb, denp,
            stage_sp, redv, outv):
        c = lax.axis_index("c")
        s = lax.axis_index("s")
        base = (c * NS + s) * rpw

        pltpu.sync_copy(ssrc_hbm, ssrc_v)
        pltpu.sync_copy(sdst_hbm, sdst_v)
        pltpu.sync_copy(srel_hbm, srel_v)
        pltpu.sync_copy(src_hbm.at[pl.ds(base, rpw), :], srcb)
        pltpu.sync_copy(dst_hbm.at[pl.ds(base, rpw), :], dstb)
        pltpu.sync_copy(et_hbm.at[pl.ds(base, rpw), :], etb)

        def zero_body(i, carry):
            denp[pl.ds(i * L, L)] = jnp.zeros((L,), jnp.float32)
            return carry
        lax.fori_loop(0, npad // L, zero_body, None)

        def chunk_body(j, carry):
            @pl.when(base + j < rows)
            def _():
                def grp_body(i, c2):
                    sl = pl.ds(i * L, L)
                    si = srcb[j, sl]
                    di = dstb[j, sl]
                    ti = etb[j, sl]
                    ev = (plsc.load_gather(ssrc_v, [si])
                          + plsc.load_gather(sdst_v, [di])
                          + plsc.load_gather(srel_v, [ti]))
                    ev = jnp.where(ev >= 0.0, ev, ev * jnp.float32(0.01))
                    ee = jnp.exp(ev)
                    eeb[j, sl] = ee
                    plsc.addupdate_scatter(denp, [di], ee)
                    return c2
                lax.fori_loop(0, CH // L, grp_body, None)
            return carry
        lax.fori_loop(0, rpw, chunk_body, None)

        pltpu.sync_copy(eeb, ee_hbm.at[pl.ds(base, rpw), :])

        pltpu.sync_copy(denp, stage_sp.at[s])
        plsc.subcore_barrier()
        pltpu.sync_copy(stage_sp.at[:, pl.ds(s * nband, nband)], redv)

        def red_body(i, carry):
            sl = pl.ds(i * L, L)
            acc = redv[0, sl]
            for k in range(1, NS):
                acc = acc + redv[k, sl]
            outv[sl] = acc
            return carry
        lax.fori_loop(0, nband // L, red_body, None)
        pltpu.sync_copy(outv, den_hbm.at[pl.ds(c * npad + s * nband, nband)])

    ee2d, den_part = _k2(s_src, s_dst, s_rel, src2d, dst2d, et2d)

    BB = 16
    nblk = rpw // BB

    @functools.partial(
        pl.kernel,
        out_type=jax.ShapeDtypeStruct((NC, n, d), jnp.float32),
        mesh=mesh,
        compiler_params=pltpu.CompilerParams(needs_layout_passes=False),
        scratch_types=[
            pltpu.VMEM((npad,), jnp.float32),
            pltpu.VMEM((npad,), jnp.float32),
            pltpu.VMEM((BB, CH), jnp.int32),
            pltpu.VMEM((BB, CH), jnp.int32),
            pltpu.VMEM((BB, CH), jnp.float32),
            pltpu.VMEM((CH,), jnp.float32),
            pltpu.VMEM((CH, d), jnp.float32),
            pltpu.VMEM_SHARED((n, d), jnp.float32),
            pltpu.SemaphoreType.DMA,
        ],
    )
    def _k3(x_hbm, src_hbm, dst_hbm, ee_hbm, den_hbm, out_hbm,
            denb, dent, srcb, dstb, eeb, alb, rowb, hout_sp, sem):
        c = lax.axis_index("c")
        s = lax.axis_index("s")
        base = (c * NS + s) * rpw

        pltpu.sync_copy(den_hbm.at[pl.ds(0, npad)], denb)
        pltpu.sync_copy(den_hbm.at[pl.ds(npad, npad)], dent)

        def add_body(i, carry):
            sl = pl.ds(i * L, L)
            denb[sl] = denb[sl] + dent[sl]
            return carry
        lax.fori_loop(0, npad // L, add_body, None)

        def zrow_body(i, carry):
            rowb[i // (d // L), pl.ds((i % (d // L)) * L, L)] = (
                jnp.zeros((L,), jnp.float32))
            return carry
        lax.fori_loop(0, CH * d // (L * L), zrow_body, None)
        band = pl.multiple_of(jnp.minimum(s * hband, n - hband), 16)
        nfull = hband // CH
        for k in range(nfull):
            pltpu.sync_copy(rowb, hout_sp.at[pl.ds(band + k * CH, CH), :])
        rem = hband - nfull * CH
        if rem:
            pltpu.sync_copy(rowb.at[pl.ds(0, rem), :],
                            hout_sp.at[pl.ds(band + nfull * CH, rem), :])
        plsc.subcore_barrier()

        def blk_body(b, carry):
            brow = base + b * BB
            pltpu.sync_copy(src_hbm.at[pl.ds(brow, BB), :], srcb)
            pltpu.sync_copy(dst_hbm.at[pl.ds(brow, BB), :], dstb)
            pltpu.sync_copy(ee_hbm.at[pl.ds(brow, BB), :], eeb)

            def chunk_body(j, carry2):
                @pl.when(brow + j < rows)
                def _():
                    desc = pltpu.async_copy(x_hbm.at[srcb.at[j]], rowb, sem)

                    def alpha_body(i, c2):
                        sl = pl.ds(i * L, L)
                        di = dstb[j, sl]
                        den = plsc.load_gather(denb, [di])
                        alb[sl] = eeb[j, sl] / den
                        return c2
                    lax.fori_loop(0, CH // L, alpha_body, None)

                    desc.wait()

                    def scale_body(rr, c2):
                        for u in range(4):
                            r2 = rr * 4 + u
                            spl = plsc.load_gather(
                                alb, [jnp.full((L,), r2, jnp.int32)])
                            for g in range(d // L):
                                sl = pl.ds(g * L, L)
                                rowb[r2, sl] = rowb[r2, sl] * spl
                        return c2
                    lax.fori_loop(0, CH // 4, scale_body, None)

                    pltpu.sync_copy(rowb, hout_sp.at[dstb.at[j]], add=True)
                return carry2
            lax.fori_loop(0, BB, chunk_body, None)
            return carry
        lax.fori_loop(0, nblk, blk_body, None)

        plsc.subcore_barrier()
        pltpu.sync_copy(hout_sp.at[pl.ds(band, hband), :],
                        out_hbm.at[c, pl.ds(band, hband), :])

    h_part = _k3(x, src2d, dst2d, ee2d, den_part)

    h_out = pl.pallas_call(
        _k4_combine,
        out_shape=jax.ShapeDtypeStruct((n, d), jnp.float32),
    )(h_part)
    return h_out
